# baseline (speedup 1.0000x reference)
import jax
import jax.numpy as jnp
from jax import lax
from jax.experimental import pallas as pl
from jax.experimental.pallas import tpu as pltpu
from jax.experimental.pallas import tpu_sc as plsc

NC = 2
NS = 16
L = 16
NW = NC * NS

KP = 4


def _emb_kernel(xi_hbm, tab_hbm, pos_hbm, out_hbm, idx_v,
                a0, a1, p0, p1, o0, o1,
                sg0, sg1, sp0, sp1, so0, so1):
    seq, d = pos_hbm.shape
    nb = out_hbm.shape[0] // seq
    per_w = seq // NW
    rows = nb * KP
    chunks = per_w // KP
    rounds = chunks // 2

    a = (a0, a1)
    p = (p0, p1)
    o = (o0, o1)
    sg = (sg0, sg1)
    sp = (sp0, sp1)
    so = (so0, so1)

    wid = lax.axis_index("s") * NC + lax.axis_index("c")
    pos_base = wid * per_w

    pltpu.sync_copy(xi_hbm.at[pl.ds(pos_base * nb, per_w * nb)], idx_v)

    def start_gather(c, b):
        pltpu.async_copy(tab_hbm.at[idx_v.at[pl.ds(c * rows, rows)]],
                         a[b], sg[b])

    def start_pos(c, b):
        pltpu.async_copy(pos_hbm.at[pl.ds(pos_base + c * KP, KP)], p[b], sp[b])

    def start_out(c, b):
        for g in range(nb):
            pltpu.async_copy(
                o[b].at[pl.ds(g * KP, KP)],
                out_hbm.at[pl.ds(g * seq + pos_base + c * KP, KP)], so[b])

    def wait_gather(c, b):
        pltpu.make_async_copy(tab_hbm.at[idx_v.at[pl.ds(c * rows, rows)]],
                              a[b], sg[b]).wait()

    def wait_pos(c, b):
        pltpu.make_async_copy(pos_hbm.at[pl.ds(pos_base + c * KP, KP)],
                              p[b], sp[b]).wait()

    def wait_out(c, b):
        for g in range(nb):
            pltpu.make_async_copy(
                o[b].at[pl.ds(g * KP, KP)],
                out_hbm.at[pl.ds(g * seq + pos_base + c * KP, KP)],
                so[b]).wait()

    def compute(b):
        def row_body(k, carry):
            for g in range(nb):
                m = g * KP + k
                for j in range(d // L):
                    sl = pl.ds(j * L, L)
                    o[b][m, sl] = a[b][m, sl] + p[b][k, sl]
            return carry
        lax.fori_loop(0, KP, row_body, 0)

    for b in range(2):
        start_gather(b, b)
        start_pos(b, b)

    def round_body(r, carry):
        for b in range(2):
            c = 2 * r + b
            wait_gather(c, b)
            wait_pos(c, b)

            @pl.when(r >= 1)
            def _():
                wait_out(c - 2, b)

            compute(b)
            start_out(c, b)
            start_gather(c + 2, b)
            start_pos(c + 2, b)
        return carry

    lax.fori_loop(0, rounds - 1, round_body, 0)

    for b in range(2):
        c = chunks - 2 + b
        wait_gather(c, b)
        wait_pos(c, b)
        wait_out(c - 2, b)
        compute(b)
        start_out(c, b)
    for b in range(2):
        wait_out(chunks - 2 + b, b)


def kernel(x, token_table, pos_table):
    b, s = x.shape
    v, d = token_table.shape
    n = b * s
    per_w = s // NW
    chunks = per_w // KP
    xi = (x.reshape(b, NW, chunks, KP)
           .transpose(1, 2, 0, 3)
           .reshape(n)
           .astype(jnp.int32))

    mesh = plsc.VectorSubcoreMesh(core_axis_name="c", subcore_axis_name="s",
                                  num_cores=NC, num_subcores=NS)
    out = pl.kernel(
        _emb_kernel,
        out_type=jax.ShapeDtypeStruct((n, d), jnp.float32),
        mesh=mesh,
        scratch_types=[
            pltpu.VMEM((n // NW,), jnp.int32),
            pltpu.VMEM((b * KP, d), jnp.float32),
            pltpu.VMEM((b * KP, d), jnp.float32),
            pltpu.VMEM((KP, d), jnp.float32),
            pltpu.VMEM((KP, d), jnp.float32),
            pltpu.VMEM((b * KP, d), jnp.float32),
            pltpu.VMEM((b * KP, d), jnp.float32),
            pltpu.SemaphoreType.DMA,
            pltpu.SemaphoreType.DMA,
            pltpu.SemaphoreType.DMA,
            pltpu.SemaphoreType.DMA,
            pltpu.SemaphoreType.DMA,
            pltpu.SemaphoreType.DMA,
        ],
    )(xi, token_table, pos_table)
    return out.reshape(b, s, d)


# speedup vs baseline: 1.3978x; 1.3978x over previous
import jax
import jax.numpy as jnp
from jax import lax
from jax.experimental import pallas as pl
from jax.experimental.pallas import tpu as pltpu
from jax.experimental.pallas import tpu_sc as plsc

NC = 2
NS = 16
L = 16
NW = NC * NS

KP = 8


def _emb_kernel(xi_hbm, tab_hbm, pos_hbm, out_hbm, idx_v,
                a0, a1, p0, p1, o0, o1,
                sg0, sg1, sp0, sp1, so0, so1):
    seq, d = pos_hbm.shape
    nb = out_hbm.shape[0] // seq
    per_w = seq // NW
    rows = nb * KP
    chunks = per_w // KP
    rounds = chunks // 2

    a = (a0, a1)
    p = (p0, p1)
    o = (o0, o1)
    sg = (sg0, sg1)
    sp = (sp0, sp1)
    so = (so0, so1)

    wid = lax.axis_index("s") * NC + lax.axis_index("c")
    pos_base = wid * per_w

    pltpu.sync_copy(xi_hbm.at[pl.ds(pos_base * nb, per_w * nb)], idx_v)

    def start_gather(c, b):
        pltpu.async_copy(tab_hbm.at[idx_v.at[pl.ds(c * rows, rows)]],
                         a[b], sg[b])

    def start_pos(c, b):
        pltpu.async_copy(pos_hbm.at[pl.ds(pos_base + c * KP, KP)], p[b], sp[b])

    def start_out(c, b):
        for g in range(nb):
            pltpu.async_copy(
                o[b].at[pl.ds(g * KP, KP)],
                out_hbm.at[pl.ds(g * seq + pos_base + c * KP, KP)], so[b])

    def wait_gather(c, b):
        pltpu.make_async_copy(tab_hbm.at[idx_v.at[pl.ds(c * rows, rows)]],
                              a[b], sg[b]).wait()

    def wait_pos(c, b):
        pltpu.make_async_copy(pos_hbm.at[pl.ds(pos_base + c * KP, KP)],
                              p[b], sp[b]).wait()

    def wait_out(c, b):
        for g in range(nb):
            pltpu.make_async_copy(
                o[b].at[pl.ds(g * KP, KP)],
                out_hbm.at[pl.ds(g * seq + pos_base + c * KP, KP)],
                so[b]).wait()

    def compute(b):
        def row_body(k, carry):
            for g in range(nb):
                m = g * KP + k
                for j in range(d // L):
                    sl = pl.ds(j * L, L)
                    o[b][m, sl] = a[b][m, sl] + p[b][k, sl]
            return carry
        lax.fori_loop(0, KP, row_body, 0)

    for b in range(2):
        start_gather(b, b)
        start_pos(b, b)

    def round_body(r, carry):
        for b in range(2):
            c = 2 * r + b
            wait_gather(c, b)
            wait_pos(c, b)

            @pl.when(r >= 1)
            def _():
                wait_out(c - 2, b)

            compute(b)
            start_gather(c + 2, b)
            start_out(c, b)
            start_pos(c + 2, b)
        return carry

    lax.fori_loop(0, rounds - 1, round_body, 0)

    for b in range(2):
        c = chunks - 2 + b
        wait_gather(c, b)
        wait_pos(c, b)
        wait_out(c - 2, b)
        compute(b)
        start_out(c, b)
    for b in range(2):
        wait_out(chunks - 2 + b, b)


def kernel(x, token_table, pos_table):
    b, s = x.shape
    v, d = token_table.shape
    n = b * s
    per_w = s // NW
    chunks = per_w // KP
    xi = (x.reshape(b, NW, chunks, KP)
           .transpose(1, 2, 0, 3)
           .reshape(n)
           .astype(jnp.int32))

    mesh = plsc.VectorSubcoreMesh(core_axis_name="c", subcore_axis_name="s",
                                  num_cores=NC, num_subcores=NS)
    out = pl.kernel(
        _emb_kernel,
        out_type=jax.ShapeDtypeStruct((n, d), jnp.float32),
        mesh=mesh,
        scratch_types=[
            pltpu.VMEM((n // NW,), jnp.int32),
            pltpu.VMEM((b * KP, d), jnp.float32),
            pltpu.VMEM((b * KP, d), jnp.float32),
            pltpu.VMEM((KP, d), jnp.float32),
            pltpu.VMEM((KP, d), jnp.float32),
            pltpu.VMEM((b * KP, d), jnp.float32),
            pltpu.VMEM((b * KP, d), jnp.float32),
            pltpu.SemaphoreType.DMA,
            pltpu.SemaphoreType.DMA,
            pltpu.SemaphoreType.DMA,
            pltpu.SemaphoreType.DMA,
            pltpu.SemaphoreType.DMA,
            pltpu.SemaphoreType.DMA,
        ],
    )(xi, token_table, pos_table)
    return out.reshape(b, s, d)


# speedup vs baseline: 1.4250x; 1.0194x over previous
import jax
import jax.numpy as jnp
from jax import lax
from jax.experimental import pallas as pl
from jax.experimental.pallas import tpu as pltpu
from jax.experimental.pallas import tpu_sc as plsc

NC = 2
NS = 16
L = 16
NW = NC * NS

KP = 8


def _emb_kernel(xi_hbm, tab_hbm, pos_hbm, out_hbm, idx_v,
                a0, a1, p0, p1, o0, o1,
                sg0, sg1, sp0, sp1, so0, so1):
    seq, d = pos_hbm.shape
    nb = out_hbm.shape[0] // seq
    per_w = seq // NW
    rows = nb * KP
    chunks = per_w // KP
    rounds = chunks // 2

    a = (a0, a1)
    p = (p0, p1)
    o = (o0, o1)
    sg = (sg0, sg1)
    sp = (sp0, sp1)
    so = (so0, so1)

    wid = lax.axis_index("s") * NC + lax.axis_index("c")
    pos_base = wid * per_w

    pltpu.sync_copy(xi_hbm.at[pl.ds(pos_base * nb, per_w * nb)], idx_v)

    def start_gather(c, b):
        pltpu.async_copy(tab_hbm.at[idx_v.at[pl.ds(c * rows, rows)]],
                         a[b], sg[b])

    def start_pos(c, b):
        pltpu.async_copy(pos_hbm.at[pl.ds(pos_base + c * KP, KP)], p[b], sp[b])

    def start_out(c, b, gs):
        for g in gs:
            pltpu.async_copy(
                o[b].at[pl.ds(g * KP, KP)],
                out_hbm.at[pl.ds(g * seq + pos_base + c * KP, KP)], so[b])

    def wait_gather(c, b):
        pltpu.make_async_copy(tab_hbm.at[idx_v.at[pl.ds(c * rows, rows)]],
                              a[b], sg[b]).wait()

    def wait_pos(c, b):
        pltpu.make_async_copy(pos_hbm.at[pl.ds(pos_base + c * KP, KP)],
                              p[b], sp[b]).wait()

    def wait_out(c, b):
        for g in range(nb):
            pltpu.make_async_copy(
                o[b].at[pl.ds(g * KP, KP)],
                out_hbm.at[pl.ds(g * seq + pos_base + c * KP, KP)],
                so[b]).wait()

    def compute(b, gs):
        def row_body(k, carry):
            for g in gs:
                m = g * KP + k
                for j in range(d // L):
                    sl = pl.ds(j * L, L)
                    o[b][m, sl] = a[b][m, sl] + p[b][k, sl]
            return carry
        lax.fori_loop(0, KP, row_body, 0)

    for b in range(2):
        start_gather(b, b)
        start_pos(b, b)

    def round_body(r, carry):
        for b in range(2):
            c = 2 * r + b
            wait_gather(c, b)
            wait_pos(c, b)

            @pl.when(r >= 1)
            def _():
                wait_out(c - 2, b)

            half = nb // 2
            compute(b, tuple(range(half)))
            start_gather(c + 2, b)
            start_out(c, b, tuple(range(half)))
            compute(b, tuple(range(half, nb)))
            start_out(c, b, tuple(range(half, nb)))
            start_pos(c + 2, b)
        return carry

    lax.fori_loop(0, rounds - 1, round_body, 0)

    for b in range(2):
        c = chunks - 2 + b
        wait_gather(c, b)
        wait_pos(c, b)
        wait_out(c - 2, b)
        compute(b, tuple(range(nb)))
        start_out(c, b, tuple(range(nb)))
    for b in range(2):
        wait_out(chunks - 2 + b, b)


def kernel(x, token_table, pos_table):
    b, s = x.shape
    v, d = token_table.shape
    n = b * s
    per_w = s // NW
    chunks = per_w // KP
    xi = (x.reshape(b, NW, chunks, KP)
           .transpose(1, 2, 0, 3)
           .reshape(n)
           .astype(jnp.int32))

    mesh = plsc.VectorSubcoreMesh(core_axis_name="c", subcore_axis_name="s",
                                  num_cores=NC, num_subcores=NS)
    out = pl.kernel(
        _emb_kernel,
        out_type=jax.ShapeDtypeStruct((n, d), jnp.float32),
        mesh=mesh,
        scratch_types=[
            pltpu.VMEM((n // NW,), jnp.int32),
            pltpu.VMEM((b * KP, d), jnp.float32),
            pltpu.VMEM((b * KP, d), jnp.float32),
            pltpu.VMEM((KP, d), jnp.float32),
            pltpu.VMEM((KP, d), jnp.float32),
            pltpu.VMEM((b * KP, d), jnp.float32),
            pltpu.VMEM((b * KP, d), jnp.float32),
            pltpu.SemaphoreType.DMA,
            pltpu.SemaphoreType.DMA,
            pltpu.SemaphoreType.DMA,
            pltpu.SemaphoreType.DMA,
            pltpu.SemaphoreType.DMA,
            pltpu.SemaphoreType.DMA,
        ],
    )(xi, token_table, pos_table)
    return out.reshape(b, s, d)


# speedup vs baseline: 1.4398x; 1.0104x over previous
import jax
import jax.numpy as jnp
from jax import lax
from jax.experimental import pallas as pl
from jax.experimental.pallas import tpu as pltpu
from jax.experimental.pallas import tpu_sc as plsc

NC = 2
NS = 16
L = 16
NW = NC * NS

KP = 8


def _emb_kernel(xi_hbm, tab_hbm, pos_hbm, out_hbm, idx_v,
                a0, a1, p0, p1, o0, o1,
                sg0, sg1, sp0, sp1, so0, so1):
    seq, d = pos_hbm.shape
    nb = out_hbm.shape[0] // seq
    per_w = seq // NW
    rows = nb * KP
    chunks = per_w // KP
    rounds = chunks // 2

    a = (a0, a1)
    p = (p0, p1)
    o = (o0, o1)
    sg = (sg0, sg1)
    sp = (sp0, sp1)
    so = (so0, so1)

    wid = lax.axis_index("s") * NC + lax.axis_index("c")
    pos_base = wid * per_w

    pltpu.sync_copy(xi_hbm.at[pl.ds(pos_base * nb, per_w * nb)], idx_v)

    def start_gather(c, b):
        pltpu.async_copy(tab_hbm.at[idx_v.at[pl.ds(c * rows, rows)]],
                         a[b], sg[b])

    def start_pos(c, b):
        pltpu.async_copy(pos_hbm.at[pl.ds(pos_base + c * KP, KP)], p[b], sp[b])

    def start_out(c, b, gs):
        for g in gs:
            pltpu.async_copy(
                o[b].at[pl.ds(g * KP, KP)],
                out_hbm.at[pl.ds(g * seq + pos_base + c * KP, KP)], so[b])

    def wait_gather(c, b):
        pltpu.make_async_copy(tab_hbm.at[idx_v.at[pl.ds(c * rows, rows)]],
                              a[b], sg[b]).wait()

    def wait_pos(c, b):
        pltpu.make_async_copy(pos_hbm.at[pl.ds(pos_base + c * KP, KP)],
                              p[b], sp[b]).wait()

    def wait_out(c, b):
        for g in range(nb):
            pltpu.make_async_copy(
                o[b].at[pl.ds(g * KP, KP)],
                out_hbm.at[pl.ds(g * seq + pos_base + c * KP, KP)],
                so[b]).wait()

    def compute(b, gs):
        def row_body(k, carry):
            for g in gs:
                m = g * KP + k
                for j in range(d // L):
                    sl = pl.ds(j * L, L)
                    o[b][m, sl] = a[b][m, sl] + p[b][k, sl]
            return carry
        lax.fori_loop(0, KP, row_body, 0)

    for b in range(2):
        start_gather(b, b)
        start_pos(b, b)

    def round_body(r, carry):
        for b in range(2):
            c = 2 * r + b
            wait_gather(c, b)
            wait_pos(c, b)

            @pl.when(r >= 1)
            def _():
                wait_out(c - 2, b)

            half = nb // 2
            compute(b, tuple(range(half)))
            start_out(c, b, tuple(range(half)))
            compute(b, tuple(range(half, nb)))
            start_gather(c + 2, b)
            start_out(c, b, tuple(range(half, nb)))
            start_pos(c + 2, b)
        return carry

    lax.fori_loop(0, rounds - 1, round_body, 0)

    for b in range(2):
        c = chunks - 2 + b
        wait_gather(c, b)
        wait_pos(c, b)
        wait_out(c - 2, b)
        compute(b, tuple(range(nb)))
        start_out(c, b, tuple(range(nb)))
    for b in range(2):
        wait_out(chunks - 2 + b, b)


def kernel(x, token_table, pos_table):
    b, s = x.shape
    v, d = token_table.shape
    n = b * s
    per_w = s // NW
    chunks = per_w // KP
    xi = (x.reshape(b, NW, chunks, KP)
           .transpose(1, 2, 0, 3)
           .reshape(n)
           .astype(jnp.int32))

    mesh = plsc.VectorSubcoreMesh(core_axis_name="c", subcore_axis_name="s",
                                  num_cores=NC, num_subcores=NS)
    out = pl.kernel(
        _emb_kernel,
        out_type=jax.ShapeDtypeStruct((n, d), jnp.float32),
        mesh=mesh,
        scratch_types=[
            pltpu.VMEM((n // NW,), jnp.int32),
            pltpu.VMEM((b * KP, d), jnp.float32),
            pltpu.VMEM((b * KP, d), jnp.float32),
            pltpu.VMEM((KP, d), jnp.float32),
            pltpu.VMEM((KP, d), jnp.float32),
            pltpu.VMEM((b * KP, d), jnp.float32),
            pltpu.VMEM((b * KP, d), jnp.float32),
            pltpu.SemaphoreType.DMA,
            pltpu.SemaphoreType.DMA,
            pltpu.SemaphoreType.DMA,
            pltpu.SemaphoreType.DMA,
            pltpu.SemaphoreType.DMA,
            pltpu.SemaphoreType.DMA,
        ],
    )(xi, token_table, pos_table)
    return out.reshape(b, s, d)
